# baseline (speedup 1.0000x reference)
import math
import functools

import jax
import jax.numpy as jnp
from jax import lax
from jax.experimental import pallas as pl
from jax.experimental.pallas import tpu as pltpu
from jax.experimental.pallas import tpu_sc as plsc

N_NODES = 10000
N_EDGES = 320000
N_ENDPOINTS = 2 * N_EDGES
NPAD = 10240
NC = 2
NS = 16
NW = NC * NS
L = 16
CHUNK = N_ENDPOINTS // NW


_ZUNROLL = 16
_SUNROLL = 24

N_CTILES = N_EDGES // 128
TILES_PER_W = N_CTILES // NW
N_EXTRA = N_CTILES - TILES_PER_W * NW
COLS_PER_W = TILES_PER_W * 128


_CHUNK_TILES = (10, 10, 10, 10, 10, 10, 9, 9)
_GROUPS_PER_ROWVEC = 8


def _sc_hist_body(edges_hbm, out_hbm, idx_v, extra_v, hist_v, *sems):
    c = lax.axis_index("c")
    s = lax.axis_index("s")
    wid = s * NC + c
    base = wid * COLS_PER_W

    extra_w = jnp.minimum(wid, N_EXTRA - 1)
    cp_extra = pltpu.async_copy(
        edges_hbm.at[
            pl.ds(0, 2), pl.ds((TILES_PER_W * NW + extra_w) * 128, 128)
        ],
        extra_v,
        sems[-1],
    )

    cps = []
    col0 = 0
    for k, nt in enumerate(_CHUNK_TILES):
        ncols = nt * 128
        cps.append(
            pltpu.async_copy(
                edges_hbm.at[pl.ds(0, 2), pl.ds(base + col0, ncols)],
                idx_v.at[pl.ds(0, 2), pl.ds(col0, ncols)],
                sems[k],
            )
        )
        col0 += ncols

    zeros = jnp.zeros((L,), jnp.int32)

    def zero_body(i, carry):
        for j in range(_ZUNROLL):
            hist_v[pl.ds((i * _ZUNROLL + j) * L, L)] = zeros
        return carry

    lax.fori_loop(0, NPAD // (L * _ZUNROLL), zero_body, 0)

    ones = jnp.ones((L,), jnp.int32)

    def add_body(i, carry):
        idxs = []
        for j in range(_GROUPS_PER_ROWVEC):
            col = pl.ds((i * _GROUPS_PER_ROWVEC + j) * L, L)
            for r in range(2):
                idxs.append(idx_v[r, col])
        for idx in idxs:
            plsc.addupdate_scatter(hist_v, [idx], ones)
        return carry

    g0 = 0
    for k, nt in enumerate(_CHUNK_TILES):
        ng = nt * 128 // L // _GROUPS_PER_ROWVEC
        cps[k].wait()
        lax.fori_loop(g0, g0 + ng, add_body, 0)
        g0 += ng

    cp_extra.wait()

    @pl.when(wid < N_EXTRA)
    def _():
        idxs = []
        for r in range(2):
            for j in range(128 // L):
                idxs.append(extra_v[r, pl.ds(j * L, L)])
        for idx in idxs:
            plsc.addupdate_scatter(hist_v, [idx], ones)

    pltpu.sync_copy(hist_v, out_hbm.at[wid])


_sc_hist = functools.partial(
    pl.kernel,
    mesh=plsc.VectorSubcoreMesh(core_axis_name="c", subcore_axis_name="s"),
    out_type=jax.ShapeDtypeStruct((NW, NPAD), jnp.int32),
    scratch_types=[
        pltpu.VMEM((2, COLS_PER_W), jnp.int32),
        pltpu.VMEM((2, 128), jnp.int32),
        pltpu.VMEM((NPAD,), jnp.int32),
    ]
    + [pltpu.SemaphoreType.DMA] * (len(_CHUNK_TILES) + 1),
    compiler_params=pltpu.CompilerParams(needs_layout_passes=False),
)(_sc_hist_body)


_MEAN = 2.0 * N_EDGES / N_NODES
def _tc_stats_body(parts_ref, out_ref):
    x = parts_ref[...]
    deg = jnp.sum(x, axis=0, keepdims=True).astype(jnp.float32)
    col = lax.broadcasted_iota(jnp.int32, (1, NPAD), 1)
    centered = jnp.where(col < N_NODES, deg - _MEAN, 0.0)
    var = jnp.sum(centered * centered) / (N_NODES - 1)
    max_deg = jnp.max(deg)

    normalized_var = var / (_MEAN + 1e-8)
    clustering = jnp.minimum(jnp.float32(1.0), normalized_var * 0.1)
    clustering = jnp.where(max_deg <= 1.0, jnp.float32(0.0), clustering)

    out_ref[0] = jnp.float32(math.log(N_NODES))
    out_ref[1] = jnp.float32(math.log(N_EDGES))
    out_ref[2] = jnp.float32(_MEAN)
    out_ref[3] = clustering
    out_ref[4] = jnp.float32(math.log(N_NODES) / math.log(max(2, _MEAN)))
    out_ref[5] = jnp.float32(2.0 * N_EDGES / (N_NODES * (N_NODES - 1)))


_tc_stats = pl.pallas_call(
    _tc_stats_body,
    out_shape=jax.ShapeDtypeStruct((6,), jnp.float32),
    out_specs=pl.BlockSpec(memory_space=pltpu.SMEM),
)


def kernel(edge_index, node_features):
    del node_features
    parts = _sc_hist(edge_index)
    return _tc_stats(parts)


# speedup vs baseline: 1.0012x; 1.0012x over previous
import math
import functools

import jax
import jax.numpy as jnp
from jax import lax
from jax.experimental import pallas as pl
from jax.experimental.pallas import tpu as pltpu
from jax.experimental.pallas import tpu_sc as plsc

N_NODES = 10000
N_EDGES = 320000
N_ENDPOINTS = 2 * N_EDGES
NPAD = 10240
NC = 2
NS = 16
NW = NC * NS
L = 16


_ZUNROLL = 16

N_CTILES = N_EDGES // 128
TILES_PER_W = N_CTILES // NW
N_EXTRA = N_CTILES - TILES_PER_W * NW
COLS_PER_W = TILES_PER_W * 128


_CHUNK_TILES = (20, 20, 19, 19)
_GROUPS_PER_ROWVEC = 8


def _sc_hist_body(edges_hbm, out_hbm, idx_v, extra_v, hist_v, *sems):
    c = lax.axis_index("c")
    s = lax.axis_index("s")
    wid = s * NC + c
    base = wid * COLS_PER_W

    extra_w = jnp.minimum(wid, N_EXTRA - 1)
    cp_extra = pltpu.async_copy(
        edges_hbm.at[
            pl.ds(0, 2), pl.ds((TILES_PER_W * NW + extra_w) * 128, 128)
        ],
        extra_v,
        sems[-1],
    )

    cps = []
    col0 = 0
    for k, nt in enumerate(_CHUNK_TILES):
        ncols = nt * 128
        cps.append(
            pltpu.async_copy(
                edges_hbm.at[pl.ds(0, 2), pl.ds(base + col0, ncols)],
                idx_v.at[pl.ds(0, 2), pl.ds(col0, ncols)],
                sems[k],
            )
        )
        col0 += ncols

    zeros = jnp.zeros((L,), jnp.int32)

    def zero_body(i, carry):
        for j in range(_ZUNROLL):
            hist_v[pl.ds((i * _ZUNROLL + j) * L, L)] = zeros
        return carry

    lax.fori_loop(0, NPAD // (L * _ZUNROLL), zero_body, 0)

    ones = jnp.ones((L,), jnp.int32)

    def add_body(i, carry):
        idxs = []
        for j in range(_GROUPS_PER_ROWVEC):
            col = pl.ds((i * _GROUPS_PER_ROWVEC + j) * L, L)
            for r in range(2):
                idxs.append(idx_v[r, col])
        for idx in idxs:
            plsc.addupdate_scatter(hist_v, [idx], ones)
        return carry

    g0 = 0
    for k, nt in enumerate(_CHUNK_TILES):
        ng = nt * 128 // L // _GROUPS_PER_ROWVEC
        cps[k].wait()
        lax.fori_loop(g0, g0 + ng, add_body, 0)
        g0 += ng

    cp_extra.wait()

    @pl.when(wid < N_EXTRA)
    def _():
        idxs = []
        for r in range(2):
            for j in range(128 // L):
                idxs.append(extra_v[r, pl.ds(j * L, L)])
        for idx in idxs:
            plsc.addupdate_scatter(hist_v, [idx], ones)

    pltpu.sync_copy(hist_v, out_hbm.at[wid])


_sc_hist = functools.partial(
    pl.kernel,
    mesh=plsc.VectorSubcoreMesh(core_axis_name="c", subcore_axis_name="s"),
    out_type=jax.ShapeDtypeStruct((NW, NPAD), jnp.int32),
    scratch_types=[
        pltpu.VMEM((2, COLS_PER_W), jnp.int32),
        pltpu.VMEM((2, 128), jnp.int32),
        pltpu.VMEM((NPAD,), jnp.int32),
    ]
    + [pltpu.SemaphoreType.DMA] * (len(_CHUNK_TILES) + 1),
    compiler_params=pltpu.CompilerParams(needs_layout_passes=False),
)(_sc_hist_body)


_MEAN = 2.0 * N_EDGES / N_NODES
def _tc_stats_body(parts_ref, out_ref):
    x = parts_ref[...]
    deg = jnp.sum(x, axis=0, keepdims=True).astype(jnp.float32)
    col = lax.broadcasted_iota(jnp.int32, (1, NPAD), 1)
    centered = jnp.where(col < N_NODES, deg - _MEAN, 0.0)
    var = jnp.sum(centered * centered) / (N_NODES - 1)
    max_deg = jnp.max(deg)

    normalized_var = var / (_MEAN + 1e-8)
    clustering = jnp.minimum(jnp.float32(1.0), normalized_var * 0.1)
    clustering = jnp.where(max_deg <= 1.0, jnp.float32(0.0), clustering)

    out_ref[0] = jnp.float32(math.log(N_NODES))
    out_ref[1] = jnp.float32(math.log(N_EDGES))
    out_ref[2] = jnp.float32(_MEAN)
    out_ref[3] = clustering
    out_ref[4] = jnp.float32(math.log(N_NODES) / math.log(max(2, _MEAN)))
    out_ref[5] = jnp.float32(2.0 * N_EDGES / (N_NODES * (N_NODES - 1)))


_tc_stats = pl.pallas_call(
    _tc_stats_body,
    out_shape=jax.ShapeDtypeStruct((6,), jnp.float32),
    out_specs=pl.BlockSpec(memory_space=pltpu.SMEM),
)


def kernel(edge_index, node_features):
    del node_features
    parts = _sc_hist(edge_index)
    return _tc_stats(parts)


# speedup vs baseline: 1.0059x; 1.0047x over previous
import math
import functools

import jax
import jax.numpy as jnp
from jax import lax
from jax.experimental import pallas as pl
from jax.experimental.pallas import tpu as pltpu
from jax.experimental.pallas import tpu_sc as plsc

N_NODES = 10000
N_EDGES = 320000
N_ENDPOINTS = 2 * N_EDGES
NPAD = 10240
NC = 2
NS = 16
NW = NC * NS
L = 16


_ZUNROLL = 16

N_CTILES = N_EDGES // 128
TILES_PER_W = N_CTILES // NW
N_EXTRA = N_CTILES - TILES_PER_W * NW
COLS_PER_W = TILES_PER_W * 128


_CHUNK_TILES = (20, 20, 19, 19)
_GROUPS_PER_ROWVEC = 4


def _sc_hist_body(edges_hbm, out_hbm, idx_v, extra_v, hist_v, *sems):
    c = lax.axis_index("c")
    s = lax.axis_index("s")
    wid = s * NC + c
    base = wid * COLS_PER_W

    extra_w = jnp.minimum(wid, N_EXTRA - 1)
    cp_extra = pltpu.async_copy(
        edges_hbm.at[
            pl.ds(0, 2), pl.ds((TILES_PER_W * NW + extra_w) * 128, 128)
        ],
        extra_v,
        sems[-1],
    )

    cps = []
    col0 = 0
    for k, nt in enumerate(_CHUNK_TILES):
        ncols = nt * 128
        cps.append(
            pltpu.async_copy(
                edges_hbm.at[pl.ds(0, 2), pl.ds(base + col0, ncols)],
                idx_v.at[pl.ds(0, 2), pl.ds(col0, ncols)],
                sems[k],
            )
        )
        col0 += ncols

    zeros = jnp.zeros((L,), jnp.int32)

    def zero_body(i, carry):
        for j in range(_ZUNROLL):
            hist_v[pl.ds((i * _ZUNROLL + j) * L, L)] = zeros
        return carry

    lax.fori_loop(0, NPAD // (L * _ZUNROLL), zero_body, 0)

    ones = jnp.ones((L,), jnp.int32)

    def add_body(i, carry):
        idxs = []
        for j in range(_GROUPS_PER_ROWVEC):
            col = pl.ds((i * _GROUPS_PER_ROWVEC + j) * L, L)
            for r in range(2):
                idxs.append(idx_v[r, col])
        for idx in idxs:
            plsc.addupdate_scatter(hist_v, [idx], ones)
        return carry

    g0 = 0
    for k, nt in enumerate(_CHUNK_TILES):
        ng = nt * 128 // L // _GROUPS_PER_ROWVEC
        cps[k].wait()
        lax.fori_loop(g0, g0 + ng, add_body, 0)
        g0 += ng

    cp_extra.wait()

    @pl.when(wid < N_EXTRA)
    def _():
        idxs = []
        for r in range(2):
            for j in range(128 // L):
                idxs.append(extra_v[r, pl.ds(j * L, L)])
        for idx in idxs:
            plsc.addupdate_scatter(hist_v, [idx], ones)

    pltpu.sync_copy(hist_v, out_hbm.at[wid])


_sc_hist = functools.partial(
    pl.kernel,
    mesh=plsc.VectorSubcoreMesh(core_axis_name="c", subcore_axis_name="s"),
    out_type=jax.ShapeDtypeStruct((NW, NPAD), jnp.int32),
    scratch_types=[
        pltpu.VMEM((2, COLS_PER_W), jnp.int32),
        pltpu.VMEM((2, 128), jnp.int32),
        pltpu.VMEM((NPAD,), jnp.int32),
    ]
    + [pltpu.SemaphoreType.DMA] * (len(_CHUNK_TILES) + 1),
    compiler_params=pltpu.CompilerParams(needs_layout_passes=False),
)(_sc_hist_body)


_MEAN = 2.0 * N_EDGES / N_NODES
def _tc_stats_body(parts_ref, out_ref):
    x = parts_ref[...]
    deg = jnp.sum(x, axis=0, keepdims=True).astype(jnp.float32)
    col = lax.broadcasted_iota(jnp.int32, (1, NPAD), 1)
    centered = jnp.where(col < N_NODES, deg - _MEAN, 0.0)
    var = jnp.sum(centered * centered) / (N_NODES - 1)
    max_deg = jnp.max(deg)

    normalized_var = var / (_MEAN + 1e-8)
    clustering = jnp.minimum(jnp.float32(1.0), normalized_var * 0.1)
    clustering = jnp.where(max_deg <= 1.0, jnp.float32(0.0), clustering)

    out_ref[0] = jnp.float32(math.log(N_NODES))
    out_ref[1] = jnp.float32(math.log(N_EDGES))
    out_ref[2] = jnp.float32(_MEAN)
    out_ref[3] = clustering
    out_ref[4] = jnp.float32(math.log(N_NODES) / math.log(max(2, _MEAN)))
    out_ref[5] = jnp.float32(2.0 * N_EDGES / (N_NODES * (N_NODES - 1)))


_tc_stats = pl.pallas_call(
    _tc_stats_body,
    out_shape=jax.ShapeDtypeStruct((6,), jnp.float32),
    out_specs=pl.BlockSpec(memory_space=pltpu.SMEM),
)


def kernel(edge_index, node_features):
    del node_features
    parts = _sc_hist(edge_index)
    return _tc_stats(parts)
